# Initial kernel scaffold; baseline (speedup 1.0000x reference)
import jax
import jax.numpy as jnp
from jax.experimental import pallas as pl


def kernel(x, values, b, indices_1, indices_2):
    raise NotImplementedError("write your pallas kernel here")


# speedup vs baseline: 4.1857x; 4.1857x over previous
import functools
import jax
import jax.numpy as jnp
from jax import lax
from jax.experimental import pallas as pl
from jax.experimental.pallas import tpu as pltpu
from jax.experimental.pallas import tpu_sc as plsc

N_IN = 65536
N_OUT = 16384
B = 256
LOGN = 11
NNZ = N_OUT * LOGN

NC = 2
NS = 16
NW = NC * NS
COLS_W = N_OUT // NW
NNZ_W = COLS_W * LOGN
G = 8
NNZ_G = G * LOGN
GROUPS = COLS_W // G
LANES = 16
CHUNKS = B // LANES


@functools.partial(
    pl.kernel,
    out_type=jax.ShapeDtypeStruct((N_OUT, B), jnp.float32),
    mesh=plsc.VectorSubcoreMesh(core_axis_name="c", subcore_axis_name="s"),
    scratch_types=[
        pltpu.VMEM((NNZ_W,), jnp.int32),
        pltpu.VMEM((NNZ_W + LANES,), jnp.float32),
        pltpu.VMEM((2, NNZ_G, B), jnp.float32),
        pltpu.VMEM((G, B), jnp.float32),
        pltpu.SemaphoreType.DMA,
        pltpu.SemaphoreType.DMA,
    ],
)
def _sc_bag(xT_hbm, idx_hbm, vals_hbm, out_hbm,
            idx_v, vals_v, rows_v, acc_v, sem0, sem1):
    wid = lax.axis_index("s") * NC + lax.axis_index("c")
    nz_base = wid * NNZ_W
    col_base = wid * COLS_W
    sems = (sem0, sem1)

    pltpu.sync_copy(idx_hbm.at[pl.ds(nz_base, NNZ_W)], idx_v)
    pltpu.sync_copy(vals_hbm.at[pl.ds(nz_base, NNZ_W)], vals_v.at[pl.ds(0, NNZ_W)])

    def start_gather(g, buf):
        pltpu.async_copy(
            xT_hbm.at[idx_v.at[pl.ds(g * NNZ_G, NNZ_G)]],
            rows_v.at[buf], sems[buf])

    def wait_gather(buf):
        pltpu.make_async_copy(
            xT_hbm.at[idx_v.at[pl.ds(0, NNZ_G)]],
            rows_v.at[buf], sems[buf]).wait()

    start_gather(0, 0)
    start_gather(1, 1)

    @pl.loop(0, GROUPS, step=2)
    def _groups(g0):
        for bslot in range(2):
            g = g0 + bslot
            wait_gather(bslot)

            @pl.loop(0, G)
            def _cols(j):
                nz0 = j * LOGN
                vvec = vals_v[pl.ds(g * NNZ_G + nz0, LANES)]
                vs = [vvec[t] for t in range(LOGN)]
                for c in range(CHUNKS):
                    sl = pl.ds(c * LANES, LANES)
                    r = rows_v[bslot, nz0, sl] * vs[0]
                    for t in range(1, LOGN):
                        r = r + rows_v[bslot, nz0 + t, sl] * vs[t]
                    acc_v[j, sl] = r

            pltpu.sync_copy(acc_v, out_hbm.at[pl.ds(col_base + g * G, G)])

            @pl.when(g + 2 < GROUPS)
            def _():
                start_gather(g + 2, bslot)


def kernel(x, values, b, indices_1, indices_2):
    xT = x.T
    yT = _sc_bag(xT, indices_1, values)
    return yT.T + b


# speedup vs baseline: 4.8464x; 1.1578x over previous
import functools
import jax
import jax.numpy as jnp
from jax import lax
from jax.experimental import pallas as pl
from jax.experimental.pallas import tpu as pltpu
from jax.experimental.pallas import tpu_sc as plsc

N_IN = 65536
N_OUT = 16384
B = 256
LOGN = 11
NNZ = N_OUT * LOGN

NC = 2
NS = 16
NW = NC * NS
COLS_W = N_OUT // NW
NNZ_W = COLS_W * LOGN
G = 8
NNZ_G = G * LOGN
GROUPS = COLS_W // G
LANES = 16
CHUNKS = B // LANES


@functools.partial(
    pl.kernel,
    out_type=jax.ShapeDtypeStruct((N_OUT, B), jnp.float32),
    mesh=plsc.VectorSubcoreMesh(core_axis_name="c", subcore_axis_name="s"),
    scratch_types=[
        pltpu.VMEM((NNZ_W,), jnp.int32),
        pltpu.VMEM((NNZ_W + LANES,), jnp.float32),
        pltpu.VMEM((2, NNZ_G, B), jnp.float32),
        pltpu.VMEM((G, B), jnp.float32),
        pltpu.SemaphoreType.DMA,
        pltpu.SemaphoreType.DMA,
    ],
)
def _sc_bag(xT_hbm, idx_hbm, vals_hbm, out_hbm,
            idx_v, vals_v, rows_v, acc_v, sem0, sem1):
    wid = lax.axis_index("s") * NC + lax.axis_index("c")
    nz_base = wid * NNZ_W
    col_base = wid * COLS_W
    sems = (sem0, sem1)

    pltpu.sync_copy(idx_hbm.at[pl.ds(nz_base, NNZ_W)], idx_v)
    pltpu.sync_copy(vals_hbm.at[pl.ds(nz_base, NNZ_W)], vals_v.at[pl.ds(0, NNZ_W)])

    def start_gather(g, buf):
        pltpu.async_copy(
            xT_hbm.at[idx_v.at[pl.ds(g * NNZ_G, NNZ_G)]],
            rows_v.at[buf], sems[buf])

    def wait_gather(buf):
        pltpu.make_async_copy(
            xT_hbm.at[idx_v.at[pl.ds(0, NNZ_G)]],
            rows_v.at[buf], sems[buf]).wait()

    start_gather(0, 0)
    start_gather(1, 1)

    @pl.loop(0, GROUPS, step=2)
    def _groups(g0):
        for bslot in range(2):
            g = g0 + bslot
            wait_gather(bslot)

            @plsc.parallel_loop(0, G)
            def _cols(j):
                nz0 = j * LOGN
                vvec = vals_v[pl.ds(g * NNZ_G + nz0, LANES)]
                vs = [vvec[t] for t in range(LOGN)]
                for c in range(CHUNKS):
                    sl = pl.ds(c * LANES, LANES)
                    terms = [rows_v[bslot, nz0 + t, sl] * vs[t]
                             for t in range(LOGN)]
                    while len(terms) > 1:
                        terms = ([terms[i] + terms[i + 1]
                                  for i in range(0, len(terms) - 1, 2)]
                                 + ([terms[-1]] if len(terms) % 2 else []))
                    acc_v[j, sl] = terms[0]

            pltpu.sync_copy(acc_v, out_hbm.at[pl.ds(col_base + g * G, G)])

            @pl.when(g + 2 < GROUPS)
            def _():
                start_gather(g + 2, bslot)


def kernel(x, values, b, indices_1, indices_2):
    xT = x.T
    yT = _sc_bag(xT, indices_1, values)
    return yT.T + b
